# baseline (speedup 1.0000x reference)
import jax
import jax.numpy as jnp
from jax.experimental import pallas as pl
from jax.experimental.pallas import tpu as pltpu

_B = 2
_N = [50000, 12500, 3125, 781]
_F = [3, 16, 16, 32]
_K = 16
_Z = 8
_N3F = _N[3] * _F[3]
_P = 25088


def _bottleneck_body(h_ref, encW_ref, encb_ref, decW_ref, decb_ref, out_ref):
    h = h_ref[...]
    z = jax.lax.dot_general(h, encW_ref[...], (((1,), (1,)), ((), ())),
                            preferred_element_type=jnp.float32)
    z = z + encb_ref[...][None, :]
    h2 = jax.lax.dot_general(z, decW_ref[...], (((1,), (1,)), ((), ())),
                             preferred_element_type=jnp.float32)
    out_ref[...] = h2 + decb_ref[...][None, :]


def _bottleneck(h, enc_W, enc_b, dec_W, dec_b):
    hp = jnp.zeros((8, _P), jnp.float32).at[:_B, :_N3F].set(h)
    encWp = jnp.zeros((_Z, _P), jnp.float32).at[:, :_N3F].set(enc_W)
    decWp = jnp.zeros((_P, _Z), jnp.float32).at[:_N3F, :].set(dec_W)
    decbp = jnp.zeros((_P,), jnp.float32).at[:_N3F].set(dec_b)
    out = pl.pallas_call(
        _bottleneck_body,
        out_shape=jax.ShapeDtypeStruct((8, _P), jnp.float32),
    )(hp, encWp, enc_b, decWp, decbp)
    return out[:_B, :_N3F]


def _vcoconv(x, edge_index, edge_cos, W, b):
    src = edge_index[0]
    dst = edge_index[1]
    xs = x[:, src, :]
    Wc = jnp.einsum('ek,kio->eio', edge_cos, W)
    msg = jnp.einsum('bei,eio->beo', xs, Wc)
    out = jnp.zeros((x.shape[0], x.shape[1], W.shape[2]), dtype=x.dtype)
    out = out.at[:, dst, :].add(msg)
    return out + b


def _pool(x, idx, w):
    g = x[:, idx, :]
    return jnp.einsum('bmkc,mk->bmc', g, w)


def kernel(x, edge_index_0, edge_cos_0, edge_index_1, edge_cos_1, edge_index_2, edge_cos_2, W_enc_0, b_enc_0, W_enc_1, b_enc_1, W_enc_2, b_enc_2, W_dec_0, b_dec_0, W_dec_1, b_dec_1, W_dec_2, b_dec_2, down_idx_0, down_w_0, up_idx_0, up_w_0, down_idx_1, down_w_1, up_idx_1, up_w_1, down_idx_2, down_w_2, up_idx_2, up_w_2, enc_W, enc_b, dec_W, dec_b):
    ei = [edge_index_0, edge_index_1, edge_index_2]
    ec = [edge_cos_0, edge_cos_1, edge_cos_2]
    We = [W_enc_0, W_enc_1, W_enc_2]
    be = [b_enc_0, b_enc_1, b_enc_2]
    Wd = [W_dec_0, W_dec_1, W_dec_2]
    bd = [b_dec_0, b_dec_1, b_dec_2]
    dn_i = [down_idx_0, down_idx_1, down_idx_2]
    dn_w = [down_w_0, down_w_1, down_w_2]
    up_i = [up_idx_0, up_idx_1, up_idx_2]
    up_w = [up_w_0, up_w_1, up_w_2]

    h = x
    for i in range(3):
        h = jax.nn.relu(_vcoconv(h, ei[i], ec[i], We[i], be[i]))
        h = _pool(h, dn_i[i], dn_w[i])
    h = h.reshape(_B, _N3F)
    h = _bottleneck(h, enc_W, enc_b, dec_W, dec_b)
    h = h.reshape(_B, _N[3], _F[3])
    for i in range(2):
        lvl = 2 - i
        h = _pool(h, up_i[lvl], up_w[lvl])
        h = jax.nn.relu(_vcoconv(h, ei[lvl], ec[lvl], Wd[i], bd[i]))
    h = _pool(h, up_i[0], up_w[0])
    out = _vcoconv(h, ei[0], ec[0], Wd[2], bd[2])
    return out


# speedup vs baseline: 14.0486x; 14.0486x over previous
import functools

import jax
import jax.numpy as jnp
from jax import lax
from jax.experimental import pallas as pl
from jax.experimental.pallas import tpu as pltpu
from jax.experimental.pallas import tpu_sc as plsc

_B = 2
_N = [50000, 12500, 3125, 781]
_F = [3, 16, 16, 32]
_K = 16
_Z = 8
_N3F = _N[3] * _F[3]
_P = 25088


def _bottleneck_body(h_ref, encW_ref, encb_ref, decW_ref, decb_ref, out_ref):
    h = h_ref[...]
    z = jax.lax.dot_general(h, encW_ref[...], (((1,), (1,)), ((), ())),
                            preferred_element_type=jnp.float32)
    z = z + encb_ref[...][None, :]
    h2 = jax.lax.dot_general(z, decW_ref[...], (((1,), (1,)), ((), ())),
                             preferred_element_type=jnp.float32)
    out_ref[...] = h2 + decb_ref[...][None, :]


def _bottleneck(h, enc_W, enc_b, dec_W, dec_b):
    hp = jnp.zeros((8, _P), jnp.float32).at[:_B, :_N3F].set(h)
    encWp = jnp.zeros((_Z, _P), jnp.float32).at[:, :_N3F].set(enc_W)
    decWp = jnp.zeros((_P, _Z), jnp.float32).at[:_N3F, :].set(dec_W)
    decbp = jnp.zeros((_P,), jnp.float32).at[:_N3F].set(dec_b)
    out = pl.pallas_call(
        _bottleneck_body,
        out_shape=jax.ShapeDtypeStruct((8, _P), jnp.float32),
    )(hp, encWp, enc_b, decWp, decbp)
    return out[:_B, :_N3F]


_CB = 16
_C = _CB * 128


@functools.lru_cache(maxsize=None)
def _make_sc_scatter(Mp, R, co, n_chunks):
    mesh = plsc.VectorSubcoreMesh(core_axis_name="c", subcore_axis_name="s")
    R16 = R // 16
    rows_per_tile = Mp // 32

    @functools.partial(
        pl.kernel, mesh=mesh,
        out_type=jax.ShapeDtypeStruct((2, R, co), jnp.float32),
        compiler_params=pltpu.CompilerParams(use_tc_tiling_on_sc=False),
        scratch_types=[
            pltpu.VMEM((_C,), jnp.int32),
            pltpu.VMEM((_C, co), jnp.float32),
            pltpu.VMEM_SHARED((R, co), jnp.float32),
        ],
    )
    def k(dst_hbm, msg_hbm, zero_hbm, out_hbm, idx_v, rows_v, acc_sh):
        c = lax.axis_index("c")
        s = lax.axis_index("s")
        wid = s * 2 + c
        pltpu.sync_copy(zero_hbm.at[pl.ds(s * R16, R16)],
                        acc_sh.at[pl.ds(s * R16, R16)])
        plsc.subcore_barrier()

        def body(j, carry):
            start = wid * rows_per_tile + j * _C
            pltpu.sync_copy(dst_hbm.at[pl.ds(start, _C)], idx_v)
            pltpu.sync_copy(msg_hbm.at[pl.ds(start, _C)], rows_v)
            pltpu.sync_copy(rows_v, acc_sh.at[idx_v], add=True)
            return carry

        lax.fori_loop(0, n_chunks, body, 0)
        plsc.subcore_barrier()
        pltpu.sync_copy(acc_sh.at[pl.ds(s * R16, R16)],
                        out_hbm.at[c, pl.ds(s * R16, R16)])

    return k


def _scatter_add_flat(m2, dst2, M, R, co_pad):
    Mp = ((M + 65535) // 65536) * 65536
    pad_idx = jnp.arange(Mp - M, dtype=jnp.int32) % R
    dst2 = jnp.concatenate([dst2, pad_idx])
    m2 = jnp.zeros((Mp, co_pad), jnp.float32).at[:M].set(m2)
    zero = jnp.zeros((R, co_pad), jnp.float32)
    f = _make_sc_scatter(Mp, R, co_pad, Mp // (32 * _C))
    part = f(dst2, m2, zero)
    return part[0] + part[1]


def _scatter_add(msg, dst, n_pad, co_pad):
    B, E, co = msg.shape
    m = jnp.zeros((B, E, co_pad), jnp.float32).at[..., :co].set(msg)
    if B * n_pad * co_pad * 4 <= 4 * 1024 * 1024:
        dst2 = (dst[None, :]
                + (jnp.arange(B, dtype=jnp.int32) * n_pad)[:, None])
        out = _scatter_add_flat(m.reshape(B * E, co_pad), dst2.reshape(-1),
                                B * E, B * n_pad, co_pad)
        return out.reshape(B, n_pad, co_pad)
    outs = [_scatter_add_flat(m[b], dst, E, n_pad, co_pad)
            for b in range(B)]
    return jnp.stack(outs)


def _vcoconv(x, edge_index, edge_cos, W, b, n_pad):
    src = edge_index[0]
    dst = edge_index[1]
    xs = x[:, src, :]
    Wc = jnp.einsum('ek,kio->eio', edge_cos, W)
    msg = jnp.einsum('bei,eio->beo', xs, Wc)
    co = W.shape[2]
    co_pad = co if co % 4 == 0 else 4
    out = _scatter_add(msg, dst, n_pad, co_pad)
    return out[:, :x.shape[1], :co] + b


def _pool(x, idx, w):
    g = x[:, idx, :]
    return jnp.einsum('bmkc,mk->bmc', g, w)


def kernel(x, edge_index_0, edge_cos_0, edge_index_1, edge_cos_1, edge_index_2, edge_cos_2, W_enc_0, b_enc_0, W_enc_1, b_enc_1, W_enc_2, b_enc_2, W_dec_0, b_dec_0, W_dec_1, b_dec_1, W_dec_2, b_dec_2, down_idx_0, down_w_0, up_idx_0, up_w_0, down_idx_1, down_w_1, up_idx_1, up_w_1, down_idx_2, down_w_2, up_idx_2, up_w_2, enc_W, enc_b, dec_W, dec_b):
    ei = [edge_index_0, edge_index_1, edge_index_2]
    ec = [edge_cos_0, edge_cos_1, edge_cos_2]
    We = [W_enc_0, W_enc_1, W_enc_2]
    be = [b_enc_0, b_enc_1, b_enc_2]
    Wd = [W_dec_0, W_dec_1, W_dec_2]
    bd = [b_dec_0, b_dec_1, b_dec_2]
    dn_i = [down_idx_0, down_idx_1, down_idx_2]
    dn_w = [down_w_0, down_w_1, down_w_2]
    up_i = [up_idx_0, up_idx_1, up_idx_2]
    up_w = [up_w_0, up_w_1, up_w_2]

    n_pad = [50048, 12544, 3136]

    h = x
    for i in range(3):
        h = jax.nn.relu(_vcoconv(h, ei[i], ec[i], We[i], be[i], n_pad[i]))
        h = _pool(h, dn_i[i], dn_w[i])
    h = h.reshape(_B, _N3F)
    h = _bottleneck(h, enc_W, enc_b, dec_W, dec_b)
    h = h.reshape(_B, _N[3], _F[3])
    for i in range(2):
        lvl = 2 - i
        h = _pool(h, up_i[lvl], up_w[lvl])
        h = jax.nn.relu(_vcoconv(h, ei[lvl], ec[lvl], Wd[i], bd[i], n_pad[lvl]))
    h = _pool(h, up_i[0], up_w[0])
    out = _vcoconv(h, ei[0], ec[0], Wd[2], bd[2], n_pad[0])
    return out
